# baseline (speedup 1.0000x reference)
import functools

import jax
import jax.numpy as jnp
from jax.experimental import pallas as pl


def _ceil_to(v, m):
    return ((v + m - 1) // m) * m


def _mm_kernel(a_ref, b_ref, o_ref):
    @pl.when(pl.program_id(2) == 0)
    def _init():
        o_ref[...] = jnp.zeros_like(o_ref)

    o_ref[...] += jnp.dot(a_ref[...], b_ref[...],
                          preferred_element_type=jnp.float32)


@functools.partial(jax.jit, static_argnames=("bm", "bk", "bn"))
def _mm_call(a, b, bm, bk, bn):
    mp, kp = a.shape
    _, np_ = b.shape
    grid = (mp // bm, np_ // bn, kp // bk)
    return pl.pallas_call(
        _mm_kernel,
        grid=grid,
        in_specs=[
            pl.BlockSpec((bm, bk), lambda i, j, k: (i, k)),
            pl.BlockSpec((bk, bn), lambda i, j, k: (k, j)),
        ],
        out_specs=pl.BlockSpec((bm, bn), lambda i, j, k: (i, j)),
        out_shape=jax.ShapeDtypeStruct((mp, np_), jnp.float32),
    )(a, b)


def _mm(a, b):
    m, k = a.shape
    k2, n = b.shape
    assert k == k2
    bm = min(512, _ceil_to(m, 128))
    bk = min(512, _ceil_to(k, 128))
    bn = min(512, _ceil_to(n, 128))
    mp, kp, np_ = _ceil_to(m, bm), _ceil_to(k, bk), _ceil_to(n, bn)
    if (mp, kp) != (m, k):
        a = jnp.pad(a, ((0, mp - m), (0, kp - k)))
    if (kp, np_) != (k, n):
        b = jnp.pad(b, ((0, kp - k), (0, np_ - n)))
    out = _mm_call(a, b, bm, bk, bn)
    if (mp, np_) != (m, n):
        out = out[:m, :n]
    return out


def _mm2_kernel(a_ref, bh_ref, bl_ref, o_ref):
    @pl.when(pl.program_id(2) == 0)
    def _init():
        o_ref[...] = jnp.zeros_like(o_ref)

    o_ref[...] += (jnp.dot(a_ref[...], bh_ref[...],
                           preferred_element_type=jnp.float32) +
                   jnp.dot(a_ref[...], bl_ref[...],
                           preferred_element_type=jnp.float32))


@functools.partial(jax.jit, static_argnames=("bm", "bk", "bn"))
def _mm2_call(a, bh, bl, bm, bk, bn):
    mp, kp = a.shape
    _, np_ = bh.shape
    grid = (mp // bm, np_ // bn, kp // bk)
    return pl.pallas_call(
        _mm2_kernel,
        grid=grid,
        in_specs=[
            pl.BlockSpec((bm, bk), lambda i, j, k: (i, k)),
            pl.BlockSpec((bk, bn), lambda i, j, k: (k, j)),
            pl.BlockSpec((bk, bn), lambda i, j, k: (k, j)),
        ],
        out_specs=pl.BlockSpec((bm, bn), lambda i, j, k: (i, j)),
        out_shape=jax.ShapeDtypeStruct((mp, np_), jnp.float32),
    )(a, bh, bl)


def _mm_split(a_bf16, b_f32):
    m, k = a_bf16.shape
    _, n = b_f32.shape
    bh = b_f32.astype(jnp.bfloat16)
    bl = (b_f32 - bh.astype(jnp.float32)).astype(jnp.bfloat16)
    bm = min(512, _ceil_to(m, 128))
    bk = min(512, _ceil_to(k, 128))
    bn = min(512, _ceil_to(n, 128))
    mp, kp, np_ = _ceil_to(m, bm), _ceil_to(k, bk), _ceil_to(n, bn)
    if (mp, kp) != (m, k):
        a_bf16 = jnp.pad(a_bf16, ((0, mp - m), (0, kp - k)))
    if (kp, np_) != (k, n):
        bh = jnp.pad(bh, ((0, kp - k), (0, np_ - n)))
        bl = jnp.pad(bl, ((0, kp - k), (0, np_ - n)))
    out = _mm2_call(a_bf16, bh, bl, bm, bk, bn)
    if (mp, np_) != (m, n):
        out = out[:m, :n]
    return out


def _rowsum_kernel(a_ref, o_ref):
    @pl.when(pl.program_id(1) == 0)
    def _init():
        o_ref[...] = jnp.zeros_like(o_ref)

    o_ref[...] += jnp.sum(a_ref[...].astype(jnp.float32), axis=1,
                          keepdims=True)


@functools.partial(jax.jit, static_argnames=("bm", "bk"))
def _rowsum_call(a, bm, bk):
    mp, kp = a.shape
    return pl.pallas_call(
        _rowsum_kernel,
        grid=(mp // bm, kp // bk),
        in_specs=[pl.BlockSpec((bm, bk), lambda i, k: (i, k))],
        out_specs=pl.BlockSpec((bm, 1), lambda i, k: (i, 0)),
        out_shape=jax.ShapeDtypeStruct((mp, 1), jnp.float32),
    )(a)


def _rowsum(a):
    m, k = a.shape
    bm = min(512, _ceil_to(m, 128))
    bk = min(512, _ceil_to(k, 128))
    mp, kp = _ceil_to(m, bm), _ceil_to(k, bk)
    if (mp, kp) != (m, k):
        a = jnp.pad(a, ((0, mp - m), (0, kp - k)))
    return _rowsum_call(a, bm, bk)[:m, 0]


_KS = [2000, 1000, 500]


def kernel(x, edge_index, Wd0, bd0, Wd1, bd1, Wd2, bd2, Wd3, bd3,
           Wu0, bu0, Wu1, bu1, Wu2, bu2, p0, p1, p2):
    Wd = [Wd0, Wd1, Wd2, Wd3]
    bd = [bd0, bd1, bd2, bd3]
    Wu = [Wu0, Wu1, Wu2]
    bu = [bu0, bu1, bu2]
    p = [p0, p1, p2]

    n = x.shape[0]
    npad = _ceil_to(n, 512)
    M0 = jnp.zeros((npad, npad), jnp.bfloat16)
    M0 = M0.at[edge_index[1], edge_index[0]].add(1.0)
    deg0 = _rowsum(M0)[:n] + 2.0
    dinv0 = jax.lax.rsqrt(deg0)

    def gcn_full(xx, W, b):
        xw = _mm(xx, W)
        xws = dinv0[:, None] * xw
        y = _mm_split(M0, jnp.pad(xws, ((0, npad - n), (0, 0))))[:n]
        return dinv0[:, None] * (y + 2.0 * xws) + b

    def gcn_small(xx, M, dinv, W, b):
        xw = _mm(xx, W)
        xws = dinv[:, None] * xw
        y = _mm(M, xws)
        return dinv[:, None] * (y + 2.0 * xws) + b

    x = jax.nn.relu(gcn_full(x, Wd[0], bd[0]))
    xs = [x]
    levels = [None]
    perms = []
    M = None
    dinv = None
    for i in range(3):
        k = _KS[i]
        score = jnp.tanh((x @ p[i]) / jnp.linalg.norm(p[i]))
        perm = jax.lax.top_k(score, k)[1]
        ik = jnp.arange(k)
        if M is None:
            Mr = M0[perm, :].at[ik, perm].set(1.0)
            Mc = M0[:, perm].at[perm, ik].set(1.0)
            Mp = _mm(Mr, Mc)
        else:
            cn = M.shape[0]
            ic = jnp.arange(cn)
            Msl = M.at[ic, ic].set(1.0)
            Mp = _mm(Msl[perm, :].astype(jnp.bfloat16),
                     Msl[:, perm].astype(jnp.bfloat16))
        Mp = Mp.at[ik, ik].set(0.0)
        x = x[perm] * score[perm][:, None]
        M = Mp
        deg = _rowsum(M) + 2.0
        dinv = jax.lax.rsqrt(deg)
        x = jax.nn.relu(gcn_small(x, M, dinv, Wd[i + 1], bd[i + 1]))
        if i < 2:
            xs.append(x)
            levels.append((M, dinv))
        perms.append(perm)
    for i in range(3):
        j = 2 - i
        res = xs[j]
        up = jnp.zeros_like(res).at[perms[j]].set(x)
        x = res + up
        if levels[j] is None:
            x = gcn_full(x, Wu[i], bu[i])
        else:
            Mj, dinvj = levels[j]
            x = gcn_small(x, Mj, dinvj, Wu[i], bu[i])
        if i < 2:
            x = jax.nn.relu(x)
    return jax.nn.log_softmax(x, axis=1)


# speedup vs baseline: 1.1965x; 1.1965x over previous
import functools

import jax
import jax.numpy as jnp
from jax.experimental import pallas as pl


def _ceil_to(v, m):
    return ((v + m - 1) // m) * m


def _mm_kernel(a_ref, b_ref, o_ref):
    @pl.when(pl.program_id(2) == 0)
    def _init():
        o_ref[...] = jnp.zeros_like(o_ref)

    o_ref[...] += jnp.dot(a_ref[...], b_ref[...],
                          preferred_element_type=jnp.float32)


@functools.partial(jax.jit, static_argnames=("bm", "bk", "bn"))
def _mm_call(a, b, bm, bk, bn):
    mp, kp = a.shape
    _, np_ = b.shape
    grid = (mp // bm, np_ // bn, kp // bk)
    return pl.pallas_call(
        _mm_kernel,
        grid=grid,
        in_specs=[
            pl.BlockSpec((bm, bk), lambda i, j, k: (i, k)),
            pl.BlockSpec((bk, bn), lambda i, j, k: (k, j)),
        ],
        out_specs=pl.BlockSpec((bm, bn), lambda i, j, k: (i, j)),
        out_shape=jax.ShapeDtypeStruct((mp, np_), jnp.float32),
    )(a, b)


def _mm(a, b):
    m, k = a.shape
    k2, n = b.shape
    assert k == k2
    bm = min(512, _ceil_to(m, 128))
    bk = min(512, _ceil_to(k, 128))
    bn = min(512, _ceil_to(n, 128))
    mp, kp, np_ = _ceil_to(m, bm), _ceil_to(k, bk), _ceil_to(n, bn)
    if (mp, kp) != (m, k):
        a = jnp.pad(a, ((0, mp - m), (0, kp - k)))
    if (kp, np_) != (k, n):
        b = jnp.pad(b, ((0, kp - k), (0, np_ - n)))
    out = _mm_call(a, b, bm, bk, bn)
    if (mp, np_) != (m, n):
        out = out[:m, :n]
    return out


def _mm2_kernel(a_ref, bh_ref, bl_ref, o_ref):
    @pl.when(pl.program_id(2) == 0)
    def _init():
        o_ref[...] = jnp.zeros_like(o_ref)

    o_ref[...] += (jnp.dot(a_ref[...], bh_ref[...],
                           preferred_element_type=jnp.float32) +
                   jnp.dot(a_ref[...], bl_ref[...],
                           preferred_element_type=jnp.float32))


@functools.partial(jax.jit, static_argnames=("bm", "bk", "bn"))
def _mm2_call(a, bh, bl, bm, bk, bn):
    mp, kp = a.shape
    _, np_ = bh.shape
    grid = (mp // bm, np_ // bn, kp // bk)
    return pl.pallas_call(
        _mm2_kernel,
        grid=grid,
        in_specs=[
            pl.BlockSpec((bm, bk), lambda i, j, k: (i, k)),
            pl.BlockSpec((bk, bn), lambda i, j, k: (k, j)),
            pl.BlockSpec((bk, bn), lambda i, j, k: (k, j)),
        ],
        out_specs=pl.BlockSpec((bm, bn), lambda i, j, k: (i, j)),
        out_shape=jax.ShapeDtypeStruct((mp, np_), jnp.float32),
    )(a, bh, bl)


def _mm_split(a_bf16, b_f32):
    m, k = a_bf16.shape
    _, n = b_f32.shape
    bh = b_f32.astype(jnp.bfloat16)
    bl = (b_f32 - bh.astype(jnp.float32)).astype(jnp.bfloat16)
    bm = min(512, _ceil_to(m, 128))
    bk = min(512, _ceil_to(k, 128))
    bn = min(512, _ceil_to(n, 128))
    mp, kp, np_ = _ceil_to(m, bm), _ceil_to(k, bk), _ceil_to(n, bn)
    if (mp, kp) != (m, k):
        a_bf16 = jnp.pad(a_bf16, ((0, mp - m), (0, kp - k)))
    if (kp, np_) != (k, n):
        bh = jnp.pad(bh, ((0, kp - k), (0, np_ - n)))
        bl = jnp.pad(bl, ((0, kp - k), (0, np_ - n)))
    out = _mm2_call(a_bf16, bh, bl, bm, bk, bn)
    if (mp, np_) != (m, n):
        out = out[:m, :n]
    return out


def _rowsum_kernel(a_ref, o_ref):
    @pl.when(pl.program_id(1) == 0)
    def _init():
        o_ref[...] = jnp.zeros_like(o_ref)

    o_ref[...] += jnp.sum(a_ref[...].astype(jnp.float32), axis=1,
                          keepdims=True)


@functools.partial(jax.jit, static_argnames=("bm", "bk"))
def _rowsum_call(a, bm, bk):
    mp, kp = a.shape
    return pl.pallas_call(
        _rowsum_kernel,
        grid=(mp // bm, kp // bk),
        in_specs=[pl.BlockSpec((bm, bk), lambda i, k: (i, k))],
        out_specs=pl.BlockSpec((bm, 1), lambda i, k: (i, 0)),
        out_shape=jax.ShapeDtypeStruct((mp, 1), jnp.float32),
    )(a)


def _rowsum(a):
    m, k = a.shape
    bm = min(512, _ceil_to(m, 128))
    bk = min(512, _ceil_to(k, 128))
    mp, kp = _ceil_to(m, bm), _ceil_to(k, bk)
    if (mp, kp) != (m, k):
        a = jnp.pad(a, ((0, mp - m), (0, kp - k)))
    return _rowsum_call(a, bm, bk)[:m, 0]


_KS = [2000, 1000, 500]


def kernel(x, edge_index, Wd0, bd0, Wd1, bd1, Wd2, bd2, Wd3, bd3,
           Wu0, bu0, Wu1, bu1, Wu2, bu2, p0, p1, p2):
    Wd = [Wd0, Wd1, Wd2, Wd3]
    bd = [bd0, bd1, bd2, bd3]
    Wu = [Wu0, Wu1, Wu2]
    bu = [bu0, bu1, bu2]
    p = [p0, p1, p2]

    n = x.shape[0]
    npad = _ceil_to(n, 512)
    M0 = jnp.zeros((npad, npad), jnp.float32)
    M0 = M0.at[edge_index[1], edge_index[0]].add(1.0)
    M0b = M0.astype(jnp.bfloat16)
    deg0 = _rowsum(M0)[:n] + 2.0
    dinv0 = jax.lax.rsqrt(deg0)

    def gcn_full(xx, W, b):
        xw = _mm(xx, W)
        xws = dinv0[:, None] * xw
        y = _mm_split(M0b, jnp.pad(xws, ((0, npad - n), (0, 0))))[:n]
        return dinv0[:, None] * (y + 2.0 * xws) + b

    def gcn_small(xx, M, dinv, W, b):
        xw = _mm(xx, W)
        xws = dinv[:, None] * xw
        y = _mm(M, xws)
        return dinv[:, None] * (y + 2.0 * xws) + b

    x = jax.nn.relu(gcn_full(x, Wd[0], bd[0]))
    xs = [x]
    levels = [None]
    perms = []
    M = None
    dinv = None
    for i in range(3):
        k = _KS[i]
        score = jnp.tanh((x @ p[i]) / jnp.linalg.norm(p[i]))
        perm = jax.lax.top_k(score, k)[1]
        ik = jnp.arange(k)
        if M is None:
            Mr = M0b[perm, :].at[ik, perm].set(1.0)
            Mc = M0b[:, perm].at[perm, ik].set(1.0)
            Mp = _mm(Mr, Mc)
        else:
            cn = M.shape[0]
            ic = jnp.arange(cn)
            Msl = M.at[ic, ic].set(1.0)
            Mp = _mm(Msl[perm, :].astype(jnp.bfloat16),
                     Msl[:, perm].astype(jnp.bfloat16))
        Mp = Mp.at[ik, ik].set(0.0)
        x = x[perm] * score[perm][:, None]
        M = Mp
        deg = _rowsum(M) + 2.0
        dinv = jax.lax.rsqrt(deg)
        x = jax.nn.relu(gcn_small(x, M, dinv, Wd[i + 1], bd[i + 1]))
        if i < 2:
            xs.append(x)
            levels.append((M, dinv))
        perms.append(perm)
    for i in range(3):
        j = 2 - i
        res = xs[j]
        up = jnp.zeros_like(res).at[perms[j]].set(x)
        x = res + up
        if levels[j] is None:
            x = gcn_full(x, Wu[i], bu[i])
        else:
            Mj, dinvj = levels[j]
            x = gcn_small(x, Mj, dinvj, Wu[i], bu[i])
        if i < 2:
            x = jax.nn.relu(x)
    return jax.nn.log_softmax(x, axis=1)
